# baseline (speedup 1.0000x reference)
import jax
import jax.numpy as jnp
from jax import lax
from jax.experimental import pallas as pl
from jax.experimental.pallas import tpu as pltpu
from jax.experimental.pallas import tpu_sc as plsc

N = 10000
E = 320000
D = 128
H = 128
C_OUT = 16

HH = 64
CHUNK = 128
NT = 16
NC = 2
N_ACC = 10240
ROWS_PER_TILE = N_ACC // NT
WB_CHUNKS = ROWS_PER_TILE // CHUNK

IB = 16
CPT = -(-E // (NC * NT * CHUNK * IB)) * IB
NB = CPT // IB
E_PAD = NC * NT * CHUNK * CPT

_MESH = plsc.VectorSubcoreMesh(
    core_axis_name="c", subcore_axis_name="s", num_cores=NC)

_SC_PARAMS = pltpu.CompilerParams(use_tc_tiling_on_sc=False)


def _fill_vmem(buf, nrows, ncols, value, dtype=jnp.float32):
    lanes = 16 if dtype == jnp.float32 else 32
    vreg = jnp.full((lanes,), value, dtype)

    def _row(i, carry):
        for j in range(ncols // lanes):
            buf[i, pl.ds(j * lanes, lanes)] = vreg
        return carry

    lax.fori_loop(0, nrows, _row, 0)


NBUF = 2


def _agg_body(h, src_r, dst_r, out, src_ib, dst_ib, rows0, rows1,
              gsem0, gsem1, ssem0, ssem1, accum):
    c = lax.axis_index("c")
    s = lax.axis_index("s")
    wid = s * NC + c
    rows = (rows0, rows1)
    gsem = (gsem0, gsem1)
    ssem = (ssem0, ssem1)

    _fill_vmem(rows0, CHUNK, H, 0.0, jnp.bfloat16)
    base = s * ROWS_PER_TILE
    for k in range(WB_CHUNKS):
        pltpu.sync_copy(rows0, accum.at[pl.ds(base + k * CHUNK, CHUNK)])

    plsc.subcore_barrier()

    def _block(b, carry):
        pltpu.sync_copy(src_r.at[wid, pl.ds(b * IB, IB)], src_ib)
        pltpu.sync_copy(dst_r.at[wid, pl.ds(b * IB, IB)], dst_ib)
        g_desc = [None] * NBUF
        s_desc = [None] * NBUF
        for p in range(NBUF - 1):
            g_desc[p] = pltpu.async_copy(
                h.at[src_ib.at[p]], rows[p], gsem[p])
        for j in range(IB):
            bi = j % NBUF
            pn = j + NBUF - 1
            if pn < IB:
                pb = pn % NBUF
                if s_desc[pb] is not None:
                    s_desc[pb].wait()
                g_desc[pb] = pltpu.async_copy(
                    h.at[src_ib.at[pn]], rows[pb], gsem[pb])
            g_desc[bi].wait()
            s_desc[bi] = pltpu.async_copy(
                rows[bi], accum.at[dst_ib.at[j]], ssem[bi], add=True)
        for q in range(NBUF):
            if s_desc[q] is not None:
                s_desc[q].wait()
        return carry

    lax.fori_loop(0, NB, _block, 0)

    plsc.subcore_barrier()

    pltpu.sync_copy(accum.at[pl.ds(base, ROWS_PER_TILE)],
                    out.at[pl.ds(c * N_ACC + base, ROWS_PER_TILE)])


_agg = pl.kernel(
    _agg_body,
    out_type=jax.ShapeDtypeStruct((NC * N_ACC, H), jnp.bfloat16),
    mesh=_MESH,
    compiler_params=_SC_PARAMS,
    scratch_types=[
        pltpu.VMEM((IB, CHUNK), jnp.int32),
        pltpu.VMEM((IB, CHUNK), jnp.int32),
        pltpu.VMEM((CHUNK, H), jnp.bfloat16),
        pltpu.VMEM((CHUNK, H), jnp.bfloat16),
        pltpu.SemaphoreType.DMA,
        pltpu.SemaphoreType.DMA,
        pltpu.SemaphoreType.DMA,
        pltpu.SemaphoreType.DMA,
        pltpu.VMEM_SHARED((N_ACC, H), jnp.bfloat16),
    ],
)

DW = 16


def _deg_body(dst_r, out, dst_ib, ones_v, dsem, accum):
    c = lax.axis_index("c")
    s = lax.axis_index("s")
    wid = s * NC + c

    _fill_vmem(ones_v, CHUNK, DW, 0.0)
    base = s * ROWS_PER_TILE
    for k in range(WB_CHUNKS):
        pltpu.sync_copy(ones_v, accum.at[pl.ds(base + k * CHUNK, CHUNK)])
    _fill_vmem(ones_v, CHUNK, DW, 1.0)

    plsc.subcore_barrier()

    def _block(b, carry):
        pltpu.sync_copy(dst_r.at[wid, pl.ds(b * IB, IB)], dst_ib)
        descs = [pltpu.async_copy(ones_v, accum.at[dst_ib.at[j]], dsem,
                                  add=True)
                 for j in range(IB)]
        for d in descs:
            d.wait()
        return carry

    lax.fori_loop(0, NB, _block, 0)

    plsc.subcore_barrier()

    pltpu.sync_copy(accum.at[pl.ds(base, ROWS_PER_TILE)],
                    out.at[pl.ds(c * N_ACC + base, ROWS_PER_TILE)])


_deg = pl.kernel(
    _deg_body,
    out_type=jax.ShapeDtypeStruct((NC * N_ACC, DW), jnp.float32),
    mesh=_MESH,
    compiler_params=_SC_PARAMS,
    scratch_types=[
        pltpu.VMEM((IB, CHUNK), jnp.int32),
        pltpu.VMEM((CHUNK, DW), jnp.float32),
        pltpu.SemaphoreType.DMA,
        pltpu.VMEM_SHARED((N_ACC, DW), jnp.float32),
    ],
)


_BLK = 1000
_GRID = N // _BLK


def _inproj_body(x_ref, w_ref, b_ref, o_ref, ob_ref):
    h = lax.dot_general(x_ref[...], w_ref[...], (((1,), (1,)), ((), ())),
                        preferred_element_type=jnp.float32)
    h = jnp.maximum(h + b_ref[...], 0.0)
    o_ref[...] = h
    ob_ref[...] = h.astype(jnp.bfloat16)


def _inproj(x, w, b):
    blk = pl.BlockSpec((_BLK, H), lambda i: (i, 0))
    return pl.pallas_call(
        _inproj_body,
        grid=(_GRID,),
        in_specs=[
            pl.BlockSpec((_BLK, D), lambda i: (i, 0)),
            pl.BlockSpec((H, D), lambda i: (0, 0)),
            pl.BlockSpec((1, H), lambda i: (0, 0)),
        ],
        out_specs=[blk, blk],
        out_shape=[jax.ShapeDtypeStruct((N, H), jnp.float32),
                   jax.ShapeDtypeStruct((N, H), jnp.bfloat16)],
    )(x, w, b)


def _layer_body(h_ref, a0_ref, a1_ref, d0_ref, d1_ref, w_ref,
                b_ref, o_ref, ob_ref):
    sm = a0_ref[...].astype(jnp.float32) + a1_ref[...].astype(jnp.float32)
    deg = d0_ref[:, 0:1] + d1_ref[:, 0:1]
    inv = 1.0 / jnp.maximum(deg, 1.0)
    rst = h_ref[...] + sm * inv
    hn = lax.dot_general(rst, w_ref[...], (((1,), (1,)), ((), ())),
                         preferred_element_type=jnp.float32)
    hn = jnp.maximum(hn + b_ref[...], 0.0)
    o_ref[...] = hn
    ob_ref[...] = hn.astype(jnp.bfloat16)


def _layer(h, a0, a1, d0, d1, w, b):
    blk = pl.BlockSpec((_BLK, H), lambda i: (i, 0))
    dblk = pl.BlockSpec((_BLK, DW), lambda i: (i, 0))
    return pl.pallas_call(
        _layer_body,
        grid=(_GRID,),
        in_specs=[blk, blk, blk, dblk, dblk,
                  pl.BlockSpec((H, H), lambda i: (0, 0)),
                  pl.BlockSpec((1, H), lambda i: (0, 0))],
        out_specs=[blk, blk],
        out_shape=[jax.ShapeDtypeStruct((N, H), jnp.float32),
                   jax.ShapeDtypeStruct((N, H), jnp.bfloat16)],
    )(h, a0, a1, d0, d1, w, b)


def _outproj_body(h_ref, w_ref, b_ref, o_ref):
    o = lax.dot_general(h_ref[...], w_ref[...], (((1,), (1,)), ((), ())),
                        preferred_element_type=jnp.float32)
    o_ref[...] = o + b_ref[...]


def _outproj(h, w, b):
    return pl.pallas_call(
        _outproj_body,
        grid=(_GRID,),
        in_specs=[
            pl.BlockSpec((_BLK, H), lambda i: (i, 0)),
            pl.BlockSpec((C_OUT, H), lambda i: (0, 0)),
            pl.BlockSpec((1, C_OUT), lambda i: (0, 0)),
        ],
        out_specs=pl.BlockSpec((_BLK, C_OUT), lambda i: (i, 0)),
        out_shape=jax.ShapeDtypeStruct((N, C_OUT), jnp.float32),
    )(h, w, b)


def kernel(x, edge_index, data, W_in, b_in, W0, b0, W1, b1, W2, b2, W_out, b_out):
    src = edge_index[0].astype(jnp.int32)
    dst = edge_index[1].astype(jnp.int32)

    src_t = jnp.concatenate(
        [src, jnp.zeros((E_PAD - E,), jnp.int32)]).reshape(
            NC * NT, CPT, CHUNK)
    dst_t = jnp.concatenate(
        [dst, jnp.full((E_PAD - E,), N, jnp.int32)]).reshape(
            NC * NT, CPT, CHUNK)

    dego = _deg(dst_t)
    d0 = dego[:N]
    d1 = dego[N_ACC:N_ACC + N]

    h, hb = _inproj(x, W_in, b_in.reshape(1, H))
    for W, b in ((W0, b0), (W1, b1), (W2, b2)):
        o = _agg(hb, src_t, dst_t)
        h, hb = _layer(h, o[:N], o[N_ACC:N_ACC + N],
                       d0, d1, W, b.reshape(1, H))
    return _outproj(h, W_out, b_out.reshape(1, C_OUT))


# speedup vs baseline: 1.1627x; 1.1627x over previous
import jax
import jax.numpy as jnp
from jax import lax
from jax.experimental import pallas as pl
from jax.experimental.pallas import tpu as pltpu
from jax.experimental.pallas import tpu_sc as plsc

N = 10000
E = 320000
D = 128
H = 128
C_OUT = 16

HH = 64
CHUNK = 128
NT = 16
NC = 2
N_ACC = 10240
ROWS_PER_TILE = N_ACC // NT
WB_CHUNKS = ROWS_PER_TILE // CHUNK

IB = 32
CPT = -(-E // (NT * CHUNK * IB)) * IB
NB = CPT // IB
E_PAD = NT * CHUNK * CPT

_MESH = plsc.VectorSubcoreMesh(
    core_axis_name="c", subcore_axis_name="s", num_cores=NC)

_SC_PARAMS = pltpu.CompilerParams(use_tc_tiling_on_sc=False)


def _fill_vmem(buf, nrows, ncols, value, dtype=jnp.float32):
    lanes = 16 if dtype == jnp.float32 else 32
    vreg = jnp.full((lanes,), value, dtype)

    def _row(i, carry):
        for j in range(ncols // lanes):
            buf[i, pl.ds(j * lanes, lanes)] = vreg
        return carry

    lax.fori_loop(0, nrows, _row, 0)


NBUF = 6


def _agg_body(h0, h1, src_r, dst_r, out, src_ib, dst_ib, rows, gsem,
              ssem, accum):
    c = lax.axis_index("c")
    s = lax.axis_index("s")

    _fill_vmem(rows[0], CHUNK, HH, 0.0, jnp.bfloat16)
    base = s * ROWS_PER_TILE
    for k in range(WB_CHUNKS):
        pltpu.sync_copy(rows[0], accum.at[pl.ds(base + k * CHUNK, CHUNK)])

    plsc.subcore_barrier()

    def _run(h):
        def _block(b, carry):
            pltpu.sync_copy(src_r.at[s, pl.ds(b * IB, IB)], src_ib)
            pltpu.sync_copy(dst_r.at[s, pl.ds(b * IB, IB)], dst_ib)
            g_desc = [None] * NBUF
            s_desc = [None] * NBUF
            for p in range(NBUF - 1):
                g_desc[p] = pltpu.async_copy(
                    h.at[src_ib.at[p]], rows[p], gsem[p])
            for j in range(IB):
                bi = j % NBUF
                pn = j + NBUF - 1
                if pn < IB:
                    pb = pn % NBUF
                    if s_desc[pb] is not None:
                        s_desc[pb].wait()
                    g_desc[pb] = pltpu.async_copy(
                        h.at[src_ib.at[pn]], rows[pb], gsem[pb])
                g_desc[bi].wait()
                s_desc[bi] = pltpu.async_copy(
                    rows[bi], accum.at[dst_ib.at[j]], ssem[bi], add=True)
            for q in range(NBUF):
                if s_desc[q] is not None:
                    s_desc[q].wait()
            return carry

        lax.fori_loop(0, NB, _block, 0)

    @pl.when(c == 0)
    def _():
        _run(h0)

    @pl.when(c == 1)
    def _():
        _run(h1)

    plsc.subcore_barrier()

    pltpu.sync_copy(accum.at[pl.ds(base, ROWS_PER_TILE)],
                    out.at[pl.ds(c * N_ACC + base, ROWS_PER_TILE)])


_agg = pl.kernel(
    _agg_body,
    out_type=jax.ShapeDtypeStruct((NC * N_ACC, HH), jnp.bfloat16),
    mesh=_MESH,
    compiler_params=_SC_PARAMS,
    scratch_types=[
        pltpu.VMEM((IB, CHUNK), jnp.int32),
        pltpu.VMEM((IB, CHUNK), jnp.int32),
        [pltpu.VMEM((CHUNK, HH), jnp.bfloat16)] * NBUF,
        [pltpu.SemaphoreType.DMA] * NBUF,
        [pltpu.SemaphoreType.DMA] * NBUF,
        pltpu.VMEM_SHARED((N_ACC, HH), jnp.bfloat16),
    ],
)

DW = 16
DEG_CPT = -(-E // (NC * NT * CHUNK * IB)) * IB
DEG_NB = DEG_CPT // IB
DEG_E_PAD = NC * NT * CHUNK * DEG_CPT


def _deg_body(dst_r, out, dst_ib, ones_v, dsem, accum):
    c = lax.axis_index("c")
    s = lax.axis_index("s")
    wid = s * NC + c

    _fill_vmem(ones_v, CHUNK, DW, 0.0)
    base = s * ROWS_PER_TILE
    for k in range(WB_CHUNKS):
        pltpu.sync_copy(ones_v, accum.at[pl.ds(base + k * CHUNK, CHUNK)])
    _fill_vmem(ones_v, CHUNK, DW, 1.0)

    plsc.subcore_barrier()

    def _block(b, carry):
        pltpu.sync_copy(dst_r.at[wid, pl.ds(b * IB, IB)], dst_ib)
        descs = [pltpu.async_copy(ones_v, accum.at[dst_ib.at[j]], dsem,
                                  add=True)
                 for j in range(IB)]
        for d in descs:
            d.wait()
        return carry

    lax.fori_loop(0, DEG_NB, _block, 0)

    plsc.subcore_barrier()

    pltpu.sync_copy(accum.at[pl.ds(base, ROWS_PER_TILE)],
                    out.at[pl.ds(c * N_ACC + base, ROWS_PER_TILE)])


_deg = pl.kernel(
    _deg_body,
    out_type=jax.ShapeDtypeStruct((NC * N_ACC, DW), jnp.float32),
    mesh=_MESH,
    compiler_params=_SC_PARAMS,
    scratch_types=[
        pltpu.VMEM((IB, CHUNK), jnp.int32),
        pltpu.VMEM((CHUNK, DW), jnp.float32),
        pltpu.SemaphoreType.DMA,
        pltpu.VMEM_SHARED((N_ACC, DW), jnp.float32),
    ],
)


_BLK = 1000
_GRID = N // _BLK


def _inproj_body(x_ref, w_ref, b_ref, o0_ref, o1_ref, ob0_ref, ob1_ref):
    h = lax.dot_general(x_ref[...], w_ref[...], (((1,), (1,)), ((), ())),
                        preferred_element_type=jnp.float32)
    h = jnp.maximum(h + b_ref[...], 0.0)
    o0_ref[...] = h[:, :HH]
    o1_ref[...] = h[:, HH:]
    ob0_ref[...] = h[:, :HH].astype(jnp.bfloat16)
    ob1_ref[...] = h[:, HH:].astype(jnp.bfloat16)


def _inproj(x, w, b):
    hblk = pl.BlockSpec((_BLK, HH), lambda i: (i, 0))
    return pl.pallas_call(
        _inproj_body,
        grid=(_GRID,),
        in_specs=[
            pl.BlockSpec((_BLK, D), lambda i: (i, 0)),
            pl.BlockSpec((H, D), lambda i: (0, 0)),
            pl.BlockSpec((1, H), lambda i: (0, 0)),
        ],
        out_specs=[hblk, hblk, hblk, hblk],
        out_shape=[jax.ShapeDtypeStruct((N, HH), jnp.float32)] * 2
        + [jax.ShapeDtypeStruct((N, HH), jnp.bfloat16)] * 2,
    )(x, w, b)


def _layer_body(h0_ref, h1_ref, a0_ref, a1_ref, d0_ref, d1_ref, w_ref,
                b_ref, o0_ref, o1_ref, ob0_ref, ob1_ref):
    h = jnp.concatenate([h0_ref[...], h1_ref[...]], axis=1)
    sm = jnp.concatenate([a0_ref[...], a1_ref[...]],
                         axis=1).astype(jnp.float32)
    deg = d0_ref[:, 0:1] + d1_ref[:, 0:1]
    inv = 1.0 / jnp.maximum(deg, 1.0)
    rst = h + sm * inv
    hn = lax.dot_general(rst, w_ref[...], (((1,), (1,)), ((), ())),
                         preferred_element_type=jnp.float32)
    hn = jnp.maximum(hn + b_ref[...], 0.0)
    o0_ref[...] = hn[:, :HH]
    o1_ref[...] = hn[:, HH:]
    ob0_ref[...] = hn[:, :HH].astype(jnp.bfloat16)
    ob1_ref[...] = hn[:, HH:].astype(jnp.bfloat16)


def _layer(h0, h1, a0, a1, d0, d1, w, b):
    hblk = pl.BlockSpec((_BLK, HH), lambda i: (i, 0))
    dblk = pl.BlockSpec((_BLK, DW), lambda i: (i, 0))
    return pl.pallas_call(
        _layer_body,
        grid=(_GRID,),
        in_specs=[hblk, hblk, hblk, hblk, dblk, dblk,
                  pl.BlockSpec((H, H), lambda i: (0, 0)),
                  pl.BlockSpec((1, H), lambda i: (0, 0))],
        out_specs=[hblk, hblk, hblk, hblk],
        out_shape=[jax.ShapeDtypeStruct((N, HH), jnp.float32)] * 2
        + [jax.ShapeDtypeStruct((N, HH), jnp.bfloat16)] * 2,
    )(h0, h1, a0, a1, d0, d1, w, b)


def _outproj_body(h0_ref, h1_ref, w_ref, b_ref, o_ref):
    h = jnp.concatenate([h0_ref[...], h1_ref[...]], axis=1)
    o = lax.dot_general(h, w_ref[...], (((1,), (1,)), ((), ())),
                        preferred_element_type=jnp.float32)
    o_ref[...] = o + b_ref[...]


def _outproj(h0, h1, w, b):
    hblk = pl.BlockSpec((_BLK, HH), lambda i: (i, 0))
    return pl.pallas_call(
        _outproj_body,
        grid=(_GRID,),
        in_specs=[
            hblk, hblk,
            pl.BlockSpec((C_OUT, H), lambda i: (0, 0)),
            pl.BlockSpec((1, C_OUT), lambda i: (0, 0)),
        ],
        out_specs=pl.BlockSpec((_BLK, C_OUT), lambda i: (i, 0)),
        out_shape=jax.ShapeDtypeStruct((N, C_OUT), jnp.float32),
    )(h0, h1, w, b)


def kernel(x, edge_index, data, W_in, b_in, W0, b0, W1, b1, W2, b2, W_out, b_out):
    src = edge_index[0].astype(jnp.int32)
    dst = edge_index[1].astype(jnp.int32)

    src_t = jnp.concatenate(
        [src, jnp.zeros((E_PAD - E,), jnp.int32)]).reshape(NT, CPT, CHUNK)
    dst_t = jnp.concatenate(
        [dst, jnp.full((E_PAD - E,), N, jnp.int32)]).reshape(NT, CPT, CHUNK)
    dst_d = jnp.concatenate(
        [dst, jnp.full((DEG_E_PAD - E,), N, jnp.int32)]).reshape(
            NC * NT, DEG_CPT, CHUNK)

    dego = _deg(dst_d)
    d0 = dego[:N]
    d1 = dego[N_ACC:N_ACC + N]

    h0, h1, hb0, hb1 = _inproj(x, W_in, b_in.reshape(1, H))
    for W, b in ((W0, b0), (W1, b1), (W2, b2)):
        o = _agg(hb0, hb1, src_t, dst_t)
        h0, h1, hb0, hb1 = _layer(h0, h1, o[:N], o[N_ACC:N_ACC + N],
                                  d0, d1, W, b.reshape(1, H))
    return _outproj(h0, h1, W_out, b_out.reshape(1, C_OUT))


# speedup vs baseline: 1.2650x; 1.0880x over previous
import jax
import jax.numpy as jnp
from jax import lax
from jax.experimental import pallas as pl
from jax.experimental.pallas import tpu as pltpu
from jax.experimental.pallas import tpu_sc as plsc

N = 10000
E = 320000
D = 128
H = 128
C_OUT = 16

HH = 64
CHUNK = 128
NT = 16
NC = 2
N_ACC = 10240
ROWS_PER_TILE = N_ACC // NT
WB_CHUNKS = ROWS_PER_TILE // CHUNK

IB = 16
CPT = -(-E // (NT * CHUNK * IB)) * IB
NB = CPT // IB
E_PAD = NT * CHUNK * CPT

_MESH = plsc.VectorSubcoreMesh(
    core_axis_name="c", subcore_axis_name="s", num_cores=NC)

_SC_PARAMS = pltpu.CompilerParams(use_tc_tiling_on_sc=False)


def _fill_vmem(buf, nrows, ncols, value, dtype=jnp.float32):
    lanes = 16 if dtype == jnp.float32 else 32
    vreg = jnp.full((lanes,), value, dtype)

    def _row(i, carry):
        for j in range(ncols // lanes):
            buf[i, pl.ds(j * lanes, lanes)] = vreg
        return carry

    lax.fori_loop(0, nrows, _row, 0)


NBUF = 4


def _agg_body(h0, h1, src_r, dst_r, out, src_ib, dst_ib, rows, gsem,
              ssem, accum):
    c = lax.axis_index("c")
    s = lax.axis_index("s")

    _fill_vmem(rows[0], CHUNK, HH, 0.0, jnp.bfloat16)
    base = s * ROWS_PER_TILE
    for k in range(WB_CHUNKS):
        pltpu.sync_copy(rows[0], accum.at[pl.ds(base + k * CHUNK, CHUNK)])

    plsc.subcore_barrier()

    def _run(h):
        def _block(b, carry):
            pltpu.sync_copy(src_r.at[s, pl.ds(b * IB, IB)], src_ib)
            pltpu.sync_copy(dst_r.at[s, pl.ds(b * IB, IB)], dst_ib)
            g_desc = [None] * NBUF
            s_desc = [None] * NBUF
            for p in range(NBUF - 1):
                g_desc[p] = pltpu.async_copy(
                    h.at[src_ib.at[p]], rows[p], gsem[p])
            for j in range(IB):
                bi = j % NBUF
                pn = j + NBUF - 1
                if pn < IB:
                    pb = pn % NBUF
                    if s_desc[pb] is not None:
                        s_desc[pb].wait()
                    g_desc[pb] = pltpu.async_copy(
                        h.at[src_ib.at[pn]], rows[pb], gsem[pb])
                g_desc[bi].wait()
                s_desc[bi] = pltpu.async_copy(
                    rows[bi], accum.at[dst_ib.at[j]], ssem[bi], add=True)
            for q in range(NBUF):
                if s_desc[q] is not None:
                    s_desc[q].wait()
            return carry

        lax.fori_loop(0, NB, _block, 0)

    @pl.when(c == 0)
    def _():
        _run(h0)

    @pl.when(c == 1)
    def _():
        _run(h1)

    plsc.subcore_barrier()

    pltpu.sync_copy(accum.at[pl.ds(base, ROWS_PER_TILE)],
                    out.at[pl.ds(c * N_ACC + base, ROWS_PER_TILE)])


_agg = pl.kernel(
    _agg_body,
    out_type=jax.ShapeDtypeStruct((NC * N_ACC, HH), jnp.bfloat16),
    mesh=_MESH,
    compiler_params=_SC_PARAMS,
    scratch_types=[
        pltpu.VMEM((IB, CHUNK), jnp.int32),
        pltpu.VMEM((IB, CHUNK), jnp.int32),
        [pltpu.VMEM((CHUNK, HH), jnp.bfloat16)] * NBUF,
        [pltpu.SemaphoreType.DMA] * NBUF,
        [pltpu.SemaphoreType.DMA] * NBUF,
        pltpu.VMEM_SHARED((N_ACC, HH), jnp.bfloat16),
    ],
)

DW = 16
DEG_CPT = -(-E // (NC * NT * CHUNK * IB)) * IB
DEG_NB = DEG_CPT // IB
DEG_E_PAD = NC * NT * CHUNK * DEG_CPT


def _deg_body(dst_r, out, dst_ib, ones_v, dsem, accum):
    c = lax.axis_index("c")
    s = lax.axis_index("s")
    wid = s * NC + c

    _fill_vmem(ones_v, CHUNK, DW, 0.0)
    base = s * ROWS_PER_TILE
    for k in range(WB_CHUNKS):
        pltpu.sync_copy(ones_v, accum.at[pl.ds(base + k * CHUNK, CHUNK)])
    _fill_vmem(ones_v, CHUNK, DW, 1.0)

    plsc.subcore_barrier()

    def _block(b, carry):
        pltpu.sync_copy(dst_r.at[wid, pl.ds(b * IB, IB)], dst_ib)
        descs = [pltpu.async_copy(ones_v, accum.at[dst_ib.at[j]], dsem,
                                  add=True)
                 for j in range(IB)]
        for d in descs:
            d.wait()
        return carry

    lax.fori_loop(0, DEG_NB, _block, 0)

    plsc.subcore_barrier()

    pltpu.sync_copy(accum.at[pl.ds(base, ROWS_PER_TILE)],
                    out.at[pl.ds(c * N_ACC + base, ROWS_PER_TILE)])


_deg = pl.kernel(
    _deg_body,
    out_type=jax.ShapeDtypeStruct((NC * N_ACC, DW), jnp.float32),
    mesh=_MESH,
    compiler_params=_SC_PARAMS,
    scratch_types=[
        pltpu.VMEM((IB, CHUNK), jnp.int32),
        pltpu.VMEM((CHUNK, DW), jnp.float32),
        pltpu.SemaphoreType.DMA,
        pltpu.VMEM_SHARED((N_ACC, DW), jnp.float32),
    ],
)


_BLK = 1000
_GRID = N // _BLK


def _inproj_body(x_ref, w_ref, b_ref, o0_ref, o1_ref, ob0_ref, ob1_ref):
    h = lax.dot_general(x_ref[...], w_ref[...], (((1,), (1,)), ((), ())),
                        preferred_element_type=jnp.float32)
    h = jnp.maximum(h + b_ref[...], 0.0)
    o0_ref[...] = h[:, :HH]
    o1_ref[...] = h[:, HH:]
    ob0_ref[...] = h[:, :HH].astype(jnp.bfloat16)
    ob1_ref[...] = h[:, HH:].astype(jnp.bfloat16)


def _inproj(x, w, b):
    hblk = pl.BlockSpec((_BLK, HH), lambda i: (i, 0))
    return pl.pallas_call(
        _inproj_body,
        grid=(_GRID,),
        in_specs=[
            pl.BlockSpec((_BLK, D), lambda i: (i, 0)),
            pl.BlockSpec((H, D), lambda i: (0, 0)),
            pl.BlockSpec((1, H), lambda i: (0, 0)),
        ],
        out_specs=[hblk, hblk, hblk, hblk],
        out_shape=[jax.ShapeDtypeStruct((N, HH), jnp.float32)] * 2
        + [jax.ShapeDtypeStruct((N, HH), jnp.bfloat16)] * 2,
    )(x, w, b)


def _layer_body(h0_ref, h1_ref, a0_ref, a1_ref, d0_ref, d1_ref, w_ref,
                b_ref, o0_ref, o1_ref, ob0_ref, ob1_ref):
    h = jnp.concatenate([h0_ref[...], h1_ref[...]], axis=1)
    sm = jnp.concatenate([a0_ref[...], a1_ref[...]],
                         axis=1).astype(jnp.float32)
    deg = d0_ref[:, 0:1] + d1_ref[:, 0:1]
    inv = 1.0 / jnp.maximum(deg, 1.0)
    rst = h + sm * inv
    hn = lax.dot_general(rst, w_ref[...], (((1,), (1,)), ((), ())),
                         preferred_element_type=jnp.float32)
    hn = jnp.maximum(hn + b_ref[...], 0.0)
    o0_ref[...] = hn[:, :HH]
    o1_ref[...] = hn[:, HH:]
    ob0_ref[...] = hn[:, :HH].astype(jnp.bfloat16)
    ob1_ref[...] = hn[:, HH:].astype(jnp.bfloat16)


def _layer(h0, h1, a0, a1, d0, d1, w, b):
    hblk = pl.BlockSpec((_BLK, HH), lambda i: (i, 0))
    dblk = pl.BlockSpec((_BLK, DW), lambda i: (i, 0))
    return pl.pallas_call(
        _layer_body,
        grid=(_GRID,),
        in_specs=[hblk, hblk, hblk, hblk, dblk, dblk,
                  pl.BlockSpec((H, H), lambda i: (0, 0)),
                  pl.BlockSpec((1, H), lambda i: (0, 0))],
        out_specs=[hblk, hblk, hblk, hblk],
        out_shape=[jax.ShapeDtypeStruct((N, HH), jnp.float32)] * 2
        + [jax.ShapeDtypeStruct((N, HH), jnp.bfloat16)] * 2,
    )(h0, h1, a0, a1, d0, d1, w, b)


def _final_body(h0_ref, h1_ref, a0_ref, a1_ref, d0_ref, d1_ref, w_ref,
                b_ref, wo_ref, bo_ref, o_ref):
    h = jnp.concatenate([h0_ref[...], h1_ref[...]], axis=1)
    sm = jnp.concatenate([a0_ref[...], a1_ref[...]],
                         axis=1).astype(jnp.float32)
    deg = d0_ref[:, 0:1] + d1_ref[:, 0:1]
    inv = 1.0 / jnp.maximum(deg, 1.0)
    rst = h + sm * inv
    hn = lax.dot_general(rst, w_ref[...], (((1,), (1,)), ((), ())),
                         preferred_element_type=jnp.float32)
    hn = jnp.maximum(hn + b_ref[...], 0.0)
    o = lax.dot_general(hn, wo_ref[...], (((1,), (1,)), ((), ())),
                        preferred_element_type=jnp.float32)
    o_ref[...] = o + bo_ref[...]


def _final(h0, h1, a0, a1, d0, d1, w, b, wo, bo):
    hblk = pl.BlockSpec((_BLK, HH), lambda i: (i, 0))
    dblk = pl.BlockSpec((_BLK, DW), lambda i: (i, 0))
    return pl.pallas_call(
        _final_body,
        grid=(_GRID,),
        in_specs=[hblk, hblk, hblk, hblk, dblk, dblk,
                  pl.BlockSpec((H, H), lambda i: (0, 0)),
                  pl.BlockSpec((1, H), lambda i: (0, 0)),
                  pl.BlockSpec((C_OUT, H), lambda i: (0, 0)),
                  pl.BlockSpec((1, C_OUT), lambda i: (0, 0))],
        out_specs=pl.BlockSpec((_BLK, C_OUT), lambda i: (i, 0)),
        out_shape=jax.ShapeDtypeStruct((N, C_OUT), jnp.float32),
    )(h0, h1, a0, a1, d0, d1, w, b, wo, bo)


def kernel(x, edge_index, data, W_in, b_in, W0, b0, W1, b1, W2, b2, W_out, b_out):
    src = edge_index[0].astype(jnp.int32)
    dst = edge_index[1].astype(jnp.int32)

    src_t = jnp.concatenate(
        [src, jnp.zeros((E_PAD - E,), jnp.int32)]).reshape(NT, CPT, CHUNK)
    dst_t = jnp.concatenate(
        [dst, jnp.full((E_PAD - E,), N, jnp.int32)]).reshape(NT, CPT, CHUNK)
    dst_d = jnp.concatenate(
        [dst, jnp.full((DEG_E_PAD - E,), N, jnp.int32)]).reshape(
            NC * NT, DEG_CPT, CHUNK)

    dego = _deg(dst_d)
    d0 = dego[:N]
    d1 = dego[N_ACC:N_ACC + N]

    h0, h1, hb0, hb1 = _inproj(x, W_in, b_in.reshape(1, H))
    for W, b in ((W0, b0), (W1, b1)):
        o = _agg(hb0, hb1, src_t, dst_t)
        h0, h1, hb0, hb1 = _layer(h0, h1, o[:N], o[N_ACC:N_ACC + N],
                                  d0, d1, W, b.reshape(1, H))
    o = _agg(hb0, hb1, src_t, dst_t)
    return _final(h0, h1, o[:N], o[N_ACC:N_ACC + N], d0, d1,
                  W2, b2.reshape(1, H), W_out, b_out.reshape(1, C_OUT))
